# baseline (speedup 1.0000x reference)
import functools

import jax
import jax.numpy as jnp
from jax import lax
from jax.experimental import pallas as pl
from jax.experimental.pallas import tpu as pltpu
from jax.experimental.pallas import tpu_sc as plsc

_NC = 2
_NS = 16
_NW = _NC * _NS


def _sc_gather(table, idx):
    n = idx.shape[0]
    d = table.shape[1]
    chunk = (-((-n) // _NW)) + 7 & ~7
    n_full = n // chunk
    rem = n - n_full * chunk
    assert rem % 8 == 0
    mesh = plsc.VectorSubcoreMesh(core_axis_name="c", subcore_axis_name="s")

    @functools.partial(
        pl.kernel,
        mesh=mesh,
        out_type=jax.ShapeDtypeStruct((n, d), jnp.float32),
        scratch_types=[
            pltpu.VMEM((chunk,), jnp.int32),
            pltpu.VMEM((chunk, d), jnp.float32),
            pltpu.SemaphoreType.DMA,
        ],
    )
    def k(table_hbm, idx_hbm, out_hbm, idx_v, rows_v, sem):
        wid = lax.axis_index("s") * _NC + lax.axis_index("c")
        base = wid * chunk

        @pl.when(wid < n_full)
        def _full():
            pltpu.sync_copy(idx_hbm.at[pl.ds(base, chunk)], idx_v)
            pltpu.async_copy(table_hbm.at[idx_v], rows_v, sem).wait()
            pltpu.sync_copy(rows_v, out_hbm.at[pl.ds(base, chunk)])

        if rem:

            @pl.when(wid == n_full)
            def _tail():
                zeros = jnp.zeros((16,), jnp.int32)
                for i in range(0, chunk, 16):
                    idx_v[pl.ds(i, 16)] = zeros
                pltpu.sync_copy(
                    idx_hbm.at[pl.ds(n_full * chunk, rem)],
                    idx_v.at[pl.ds(0, rem)],
                )
                pltpu.async_copy(table_hbm.at[idx_v], rows_v, sem).wait()
                pltpu.sync_copy(
                    rows_v.at[pl.ds(0, rem)],
                    out_hbm.at[pl.ds(n_full * chunk, rem)],
                )

    return k(table, idx)


def _mm_body(d, x_ref, z_ref, o_ref):
    o_ref[...] = lax.dot_general(
        x_ref[...],
        z_ref[:, :d],
        dimension_numbers=(((1,), (1,)), ((), ())),
        preferred_element_type=jnp.float32,
    )


_BN = 128


def _tc_matmul(x, z):
    b, d = x.shape
    n = z.shape[0]
    grid = (n + _BN - 1) // _BN
    return pl.pallas_call(
        functools.partial(_mm_body, d),
        grid=(grid,),
        in_specs=[
            pl.BlockSpec((b, d), lambda j: (0, 0)),
            pl.BlockSpec((_BN, z.shape[1]), lambda j: (j, 0)),
        ],
        out_specs=pl.BlockSpec((b, _BN), lambda j: (0, j)),
        out_shape=jax.ShapeDtypeStruct((b, n), jnp.float32),
        compiler_params=pltpu.CompilerParams(
            dimension_semantics=("arbitrary",),
        ),
    )(x, z)


def kernel(x_data, encoded_labels, emb_table):
    d = emb_table.shape[1]
    idx = encoded_labels.astype(jnp.int32)
    dpad = (-d) % 128
    table = jnp.pad(emb_table, ((0, 0), (0, dpad))) if dpad else emb_table
    z_label = _sc_gather(table, idx)
    return _tc_matmul(x_data, z_label)


# speedup vs baseline: 1.1125x; 1.1125x over previous
import functools

import jax
import jax.numpy as jnp
from jax import lax
from jax.experimental import pallas as pl
from jax.experimental.pallas import tpu as pltpu
from jax.experimental.pallas import tpu_sc as plsc

_NC = 2
_NS = 16
_NW = _NC * _NS


def _sc_gather(table, idx):
    n = idx.shape[0]
    d = table.shape[1]
    chunk = next(
        c for c in range(8, n + 1, 8) if n % c == 0 and n // c <= _NW
    )
    n_active = n // chunk
    mesh = plsc.VectorSubcoreMesh(core_axis_name="c", subcore_axis_name="s")

    @functools.partial(
        pl.kernel,
        mesh=mesh,
        out_type=jax.ShapeDtypeStruct((n, d), jnp.float32),
        scratch_types=[
            pltpu.VMEM((chunk,), jnp.int32),
            pltpu.VMEM((chunk, d), jnp.float32),
            pltpu.SemaphoreType.DMA,
        ],
    )
    def k(table_hbm, idx_hbm, out_hbm, idx_v, rows_v, sem):
        wid = lax.axis_index("s") * _NC + lax.axis_index("c")
        base = wid * chunk

        @pl.when(wid < n_active)
        def _run():
            pltpu.sync_copy(idx_hbm.at[pl.ds(base, chunk)], idx_v)
            pltpu.async_copy(table_hbm.at[idx_v], rows_v, sem).wait()
            pltpu.sync_copy(rows_v, out_hbm.at[pl.ds(base, chunk)])

    return k(table, idx)


def _mm_body(d, x_ref, z_ref, o_ref):
    o_ref[...] = lax.dot_general(
        x_ref[...],
        z_ref[:, :d],
        dimension_numbers=(((1,), (1,)), ((), ())),
        preferred_element_type=jnp.float32,
    )


def _tc_matmul(x, z):
    b, d = x.shape
    n = z.shape[0]
    return pl.pallas_call(
        functools.partial(_mm_body, d),
        out_shape=jax.ShapeDtypeStruct((b, n), jnp.float32),
    )(x, z)


def kernel(x_data, encoded_labels, emb_table):
    d = emb_table.shape[1]
    idx = encoded_labels.astype(jnp.int32)
    dpad = (-d) % 128
    table = jnp.pad(emb_table, ((0, 0), (0, dpad))) if dpad else emb_table
    z_label = _sc_gather(table, idx)
    return _tc_matmul(x_data, z_label)
